# baseline (speedup 1.0000x reference)
import functools

import jax
import jax.numpy as jnp
from jax import lax
from jax.experimental import pallas as pl
from jax.experimental.pallas import tpu as pltpu
from jax.experimental.pallas import tpu_sc as plsc

N = 10000
E = 320000
D = 128
H = 128
C = 10
G = 64
ALPHA = 0.01

NC = 2
NS = 16
NW = NC * NS
CHUNK = 128
NB = 2
E_PAD = 327680
ROWS_PT = E_PAD // NW // CHUNK
SCHUNKS = ROWS_PT // NB
R_PAD = 10240
RPT = R_PAD // NS


def _sc_mesh():
    return plsc.VectorSubcoreMesh(core_axis_name="c", subcore_axis_name="s")


def _deg_body(dst_h, ones_h, zero_h, out_h, deg_sh, ones_v, didx, _sem):
    c = lax.axis_index("c")
    s = lax.axis_index("s")
    wid = s * NC + c
    pltpu.sync_copy(ones_h, ones_v)
    pltpu.sync_copy(zero_h.at[pl.ds(s * RPT, RPT)], deg_sh.at[pl.ds(s * RPT, RPT)])
    plsc.subcore_barrier()

    def body(i, carry):
        rb = wid * ROWS_PT + i * NB
        pltpu.sync_copy(dst_h.at[pl.ds(rb, NB)], didx)
        for j in range(NB):
            pltpu.sync_copy(ones_v, deg_sh.at[didx.at[j]], add=True)
        return carry

    lax.fori_loop(0, SCHUNKS, body, 0)
    plsc.subcore_barrier()
    pltpu.sync_copy(deg_sh.at[pl.ds(s * RPT, RPT)],
                    out_h.at[c, pl.ds(s * RPT, RPT)])


def _sc_degree(dst2, ones128, zeros128):
    return pl.kernel(
        _deg_body,
        out_type=jax.ShapeDtypeStruct((NC, R_PAD, H), jnp.float32),
        mesh=_sc_mesh(),
        scratch_types=[
            pltpu.VMEM_SHARED((R_PAD, H), jnp.float32),
            pltpu.VMEM((CHUNK, H), jnp.float32),
            pltpu.VMEM((NB, CHUNK), jnp.int32),
            pltpu.SemaphoreType.DMA,
        ],
    )(dst2, ones128, zeros128)


FAST_C = 1
A_ROWS = 80
B_ROWS = 2 * ROWS_PT - A_ROWS
IB = 40


def _agg_body(g_h, src_h, dst_h, zero_h, out_h, agg_sh, buf, sidx, didx,
              gsem, ssem):
    c = lax.axis_index("c")
    s = lax.axis_index("s")
    fast = c == FAST_C
    base = pl.multiple_of(jnp.where(fast, s * A_ROWS, NS * A_ROWS + s * B_ROWS), 8)
    nchunks = jnp.where(fast, A_ROWS, B_ROWS)
    pltpu.sync_copy(zero_h.at[pl.ds(s * RPT, RPT)], agg_sh.at[pl.ds(s * RPT, RPT)])
    pltpu.sync_copy(src_h.at[pl.ds(base, IB)], sidx)
    pltpu.sync_copy(dst_h.at[pl.ds(base, IB)], didx)
    plsc.subcore_barrier()
    pltpu.async_copy(g_h.at[sidx.at[0]], buf.at[0], gsem)

    def body(it, carry):
        for b in range(2):
            i = it * 2 + b
            r = lax.rem(i, IB)
            pltpu.make_async_copy(g_h.at[sidx.at[r]], buf.at[b], gsem).wait()
            @pl.when((r == IB - 1) & (i + 1 < nchunks))
            def _():
                pltpu.sync_copy(
                    src_h.at[pl.ds(pl.multiple_of(base + i + 1, 8), IB)], sidx)
            @pl.when(i > 0)
            def _():
                pltpu.make_async_copy(
                    buf.at[1 - b], agg_sh.at[didx.at[0]], ssem).wait()
            @pl.when((r == 0) & (i > 0))
            def _():
                pltpu.sync_copy(
                    dst_h.at[pl.ds(pl.multiple_of(base + i, 8), IB)], didx)
            pltpu.async_copy(buf.at[b], agg_sh.at[didx.at[r]], ssem, add=True)
            @pl.when(i + 1 < nchunks)
            def _():
                rn = lax.rem(i + 1, IB)
                pltpu.async_copy(g_h.at[sidx.at[rn]], buf.at[1 - b], gsem)
        return carry

    lax.fori_loop(0, lax.div(nchunks, 2), body, 0)
    pltpu.make_async_copy(buf.at[1], agg_sh.at[didx.at[0]], ssem).wait()
    plsc.subcore_barrier()
    pltpu.sync_copy(agg_sh.at[pl.ds(s * RPT, RPT)],
                    out_h.at[c, pl.ds(s * RPT, RPT)])


def _sc_aggregate(g, src2, dst2, zeros128):
    return pl.kernel(
        _agg_body,
        out_type=jax.ShapeDtypeStruct((NC, R_PAD, H), jnp.float32),
        mesh=_sc_mesh(),
        scratch_types=[
            pltpu.VMEM_SHARED((R_PAD, H), jnp.float32),
            pltpu.VMEM((2, CHUNK, H), jnp.float32),
            pltpu.VMEM((IB, CHUNK), jnp.int32),
            pltpu.VMEM((IB, CHUNK), jnp.int32),
            pltpu.SemaphoreType.DMA,
            pltpu.SemaphoreType.DMA,
        ],
    )(g, src2, dst2, zeros128)


BLK = 1000
NBLK = N // BLK


def _tc0_body(x_ref, w_ref, b_ref, degs_ref, dinv_ref, g_ref):
    deg = degs_ref[0, :, 0:1] + degs_ref[1, :, 0:1] + 1.0
    dinv = lax.rsqrt(deg)
    dinv_ref[...] = dinv
    h = jnp.dot(x_ref[...], w_ref[...], preferred_element_type=jnp.float32)
    g_ref[...] = (h + b_ref[...]) * dinv


def _tc0(x, w, b, degs):
    return pl.pallas_call(
        _tc0_body,
        grid=(NBLK,),
        in_specs=[
            pl.BlockSpec((BLK, D), lambda i: (i, 0)),
            pl.BlockSpec((D, H), lambda i: (0, 0)),
            pl.BlockSpec((1, H), lambda i: (0, 0)),
            pl.BlockSpec((NC, BLK, H), lambda i: (0, i, 0)),
        ],
        out_specs=[
            pl.BlockSpec((BLK, 1), lambda i: (i, 0)),
            pl.BlockSpec((BLK, H), lambda i: (i, 0)),
        ],
        out_shape=[
            jax.ShapeDtypeStruct((N, 1), jnp.float32),
            jax.ShapeDtypeStruct((N, H), jnp.float32),
        ],
    )(x, w, b, degs)


def _leaky(y):
    return jnp.where(y >= 0, y, ALPHA * y)


def _tc_mid_body(a_ref, gp_ref, dinv_ref, w_ref, b_ref, x_ref, g_ref):
    dinv = dinv_ref[...]
    y = (a_ref[0] + a_ref[1] + gp_ref[...]) * dinv
    xo = _leaky(y)
    x_ref[...] = xo
    h = jnp.dot(xo, w_ref[...], preferred_element_type=jnp.float32)
    g_ref[...] = (h + b_ref[...]) * dinv


def _tc_mid(aggs, g_prev, dinv, w, b):
    return pl.pallas_call(
        _tc_mid_body,
        grid=(NBLK,),
        in_specs=[
            pl.BlockSpec((NC, BLK, H), lambda i: (0, i, 0)),
            pl.BlockSpec((BLK, H), lambda i: (i, 0)),
            pl.BlockSpec((BLK, 1), lambda i: (i, 0)),
            pl.BlockSpec((H, H), lambda i: (0, 0)),
            pl.BlockSpec((1, H), lambda i: (0, 0)),
        ],
        out_specs=[
            pl.BlockSpec((BLK, H), lambda i: (i, 0)),
            pl.BlockSpec((BLK, H), lambda i: (i, 0)),
        ],
        out_shape=[
            jax.ShapeDtypeStruct((N, H), jnp.float32),
            jax.ShapeDtypeStruct((N, H), jnp.float32),
        ],
    )(aggs, g_prev, dinv, w, b)


def _tc_l3_body(a_ref, gp_ref, dinv_ref, x0_ref, w_ref, b_ref,
                x2_ref, x3_ref, g_ref):
    dinv = dinv_ref[...]
    y = (a_ref[0] + a_ref[1] + gp_ref[...]) * dinv
    x2 = _leaky(y)
    x2_ref[...] = x2
    x3 = x0_ref[...] + x2
    x3_ref[...] = x3
    h = jnp.dot(x3, w_ref[...], preferred_element_type=jnp.float32)
    g_ref[...] = (h + b_ref[...]) * dinv


def _tc_l3(aggs, g_prev, dinv, x0, w, b):
    return pl.pallas_call(
        _tc_l3_body,
        grid=(NBLK,),
        in_specs=[
            pl.BlockSpec((NC, BLK, H), lambda i: (0, i, 0)),
            pl.BlockSpec((BLK, H), lambda i: (i, 0)),
            pl.BlockSpec((BLK, 1), lambda i: (i, 0)),
            pl.BlockSpec((BLK, H), lambda i: (i, 0)),
            pl.BlockSpec((H, H), lambda i: (0, 0)),
            pl.BlockSpec((1, H), lambda i: (0, 0)),
        ],
        out_specs=[
            pl.BlockSpec((BLK, H), lambda i: (i, 0)),
            pl.BlockSpec((BLK, H), lambda i: (i, 0)),
            pl.BlockSpec((BLK, H), lambda i: (i, 0)),
        ],
        out_shape=[
            jax.ShapeDtypeStruct((N, H), jnp.float32),
            jax.ShapeDtypeStruct((N, H), jnp.float32),
            jax.ShapeDtypeStruct((N, H), jnp.float32),
        ],
    )(aggs, g_prev, dinv, x0, w, b)


NEG = -3.0e38


def _final_body(a_ref, gp_ref, dinv_ref, x1_ref, x2_ref, x3_ref, batch_ref,
                wm1_ref, bm1_ref, gamma_ref, beta_ref, wm2_ref, bm2_ref,
                out_ref, xt_ref):
    def xt_blk(i, carry):
        sl = pl.ds(i * BLK, BLK)
        y = (a_ref[0, sl, :] + a_ref[1, sl, :] + gp_ref[sl, :]) * dinv_ref[sl, :]
        xt_ref[sl, :] = _leaky(y)
        return carry

    lax.fori_loop(0, NBLK, xt_blk, 0)

    def pool(src_ref):
        def blk(i, acc):
            sl = pl.ds(i * BLK, BLK)
            rows = src_ref[sl, :]
            b = batch_ref[sl, :]
            glo = jnp.min(b)
            ghi = jnp.max(b)

            def gl(g, acc):
                m = b == g
                v = jnp.max(jnp.where(m, rows, NEG), axis=0, keepdims=True)
                sel = lax.broadcasted_iota(jnp.int32, (G, 1), 0) == g
                return jnp.where(sel, jnp.maximum(acc, v), acc)

            return lax.fori_loop(glo, ghi + 1, gl, acc)

        return lax.fori_loop(0, NBLK, blk, jnp.full((G, H), NEG, jnp.float32))

    pooled = jnp.concatenate(
        [pool(xt_ref), pool(x1_ref), pool(x2_ref), pool(x3_ref)], axis=1)

    h = jnp.dot(pooled, wm1_ref[...], preferred_element_type=jnp.float32)
    h = h + bm1_ref[...]
    mean = jnp.mean(h, axis=0, keepdims=True)
    var = jnp.mean((h - mean) ** 2, axis=0, keepdims=True)
    h = (h - mean) * lax.rsqrt(var + 1e-5) * gamma_ref[...] + beta_ref[...]
    h = jnp.maximum(h, 0.0)
    out_ref[...] = jnp.dot(h, wm2_ref[...], preferred_element_type=jnp.float32) + bm2_ref[...]


def _tc_final(aggs, g3, dinv, x1, x2, x3, batch2, wm1, bm1, gamma, beta, wm2, bm2):
    return pl.pallas_call(
        _final_body,
        out_shape=jax.ShapeDtypeStruct((G, C), jnp.float32),
        scratch_shapes=[pltpu.VMEM((N, H), jnp.float32)],
    )(aggs, g3, dinv, x1, x2, x3, batch2,
      wm1, bm1, gamma, beta, wm2, bm2)


def kernel(x, edge_index, batch, W0, b0, W1, b1, W2, b2, W3, b3,
           Wm1, bm1, gamma, beta, Wm2, bm2):
    order = jnp.argsort(edge_index[0])
    edge_index = edge_index[:, order]
    pad = E_PAD - E
    src2 = jnp.concatenate(
        [edge_index[0], jnp.zeros((pad,), jnp.int32)]).reshape(E_PAD // CHUNK, CHUNK)
    dst2 = jnp.concatenate(
        [edge_index[1], jnp.full((pad,), N, jnp.int32)]).reshape(E_PAD // CHUNK, CHUNK)
    zeros128 = jnp.zeros((R_PAD, H), jnp.float32)
    ones128 = jnp.ones((CHUNK, H), jnp.float32)
    batch2 = batch.reshape(N, 1)
    b0r, b1r, b2r, b3r = (v.reshape(1, H) for v in (b0, b1, b2, b3))
    bm1r = bm1.reshape(1, H)
    gammar = gamma.reshape(1, H)
    betar = beta.reshape(1, H)
    bm2r = bm2.reshape(1, C)

    degs = _sc_degree(dst2, ones128, zeros128)
    dinv, g0 = _tc0(x, W0, b0r, degs)
    a0 = _sc_aggregate(g0, src2, dst2, zeros128)
    x0, g1 = _tc_mid(a0, g0, dinv, W1, b1r)
    a1 = _sc_aggregate(g1, src2, dst2, zeros128)
    x1, g2 = _tc_mid(a1, g1, dinv, W2, b2r)
    a2 = _sc_aggregate(g2, src2, dst2, zeros128)
    x2, x3, g3 = _tc_l3(a2, g2, dinv, x0, W3, b3r)
    a3 = _sc_aggregate(g3, src2, dst2, zeros128)
    return _tc_final(a3, g3, dinv, x1, x2, x3, batch2,
                     Wm1, bm1r, gammar, betar, Wm2, bm2r)


# speedup vs baseline: 1.4973x; 1.4973x over previous
import functools

import jax
import jax.numpy as jnp
from jax import lax
from jax.experimental import pallas as pl
from jax.experimental.pallas import tpu as pltpu
from jax.experimental.pallas import tpu_sc as plsc

N = 10000
E = 320000
D = 128
H = 128
C = 10
G = 64
ALPHA = 0.01

NC = 2
NS = 16
NW = NC * NS
CHUNK = 128
NB = 2
E_PAD = 327680
ROWS_PT = E_PAD // NW // CHUNK
SCHUNKS = ROWS_PT // NB
R_PAD = 10240
RPT = R_PAD // NS


def _sc_mesh():
    return plsc.VectorSubcoreMesh(core_axis_name="c", subcore_axis_name="s")


def _deg_body(dst_h, ones_h, zero_h, out_h, deg_sh, ones_v, didx, _sem):
    c = lax.axis_index("c")
    s = lax.axis_index("s")
    wid = s * NC + c
    pltpu.sync_copy(ones_h, ones_v)
    pltpu.sync_copy(zero_h.at[pl.ds(s * RPT, RPT)], deg_sh.at[pl.ds(s * RPT, RPT)])
    plsc.subcore_barrier()

    def body(i, carry):
        rb = wid * ROWS_PT + i * NB
        pltpu.sync_copy(dst_h.at[pl.ds(rb, NB)], didx)
        for j in range(NB):
            pltpu.sync_copy(ones_v, deg_sh.at[didx.at[j]], add=True)
        return carry

    lax.fori_loop(0, SCHUNKS, body, 0)
    plsc.subcore_barrier()
    pltpu.sync_copy(deg_sh.at[pl.ds(s * RPT, RPT)],
                    out_h.at[c, pl.ds(s * RPT, RPT)])


def _sc_degree(dst2, ones128, zeros128):
    return pl.kernel(
        _deg_body,
        out_type=jax.ShapeDtypeStruct((NC, R_PAD, H), jnp.float32),
        mesh=_sc_mesh(),
        scratch_types=[
            pltpu.VMEM_SHARED((R_PAD, H), jnp.float32),
            pltpu.VMEM((CHUNK, H), jnp.float32),
            pltpu.VMEM((NB, CHUNK), jnp.int32),
            pltpu.SemaphoreType.DMA,
        ],
    )(dst2, ones128, zeros128)


FAST_C = 1
A_ROWS = 152
B_ROWS = 2 * ROWS_PT - A_ROWS
IB = 40


def _agg_body(g_h, src_h, dst_h, zero_h, out_h, agg_sh, buf, sidx, didx,
              gsem, ssem):
    c = lax.axis_index("c")
    s = lax.axis_index("s")
    fast = c == FAST_C
    base = pl.multiple_of(jnp.where(fast, s * A_ROWS, NS * A_ROWS + s * B_ROWS), 8)
    nchunks = jnp.where(fast, A_ROWS, B_ROWS)
    pltpu.sync_copy(zero_h.at[pl.ds(s * RPT, RPT)], agg_sh.at[pl.ds(s * RPT, RPT)])
    pltpu.sync_copy(src_h.at[pl.ds(base, IB)], sidx)
    pltpu.sync_copy(dst_h.at[pl.ds(base, IB)], didx)
    plsc.subcore_barrier()
    pltpu.async_copy(g_h.at[sidx.at[0]], buf.at[0], gsem)

    def body(it, carry):
        for b in range(2):
            i = it * 2 + b
            r = lax.rem(i, IB)
            pltpu.make_async_copy(g_h.at[sidx.at[r]], buf.at[b], gsem).wait()
            @pl.when((r == IB - 1) & (i + 1 < nchunks))
            def _():
                pltpu.sync_copy(
                    src_h.at[pl.ds(pl.multiple_of(base + i + 1, 8), IB)], sidx)
            @pl.when(i > 0)
            def _():
                pltpu.make_async_copy(
                    buf.at[1 - b], agg_sh.at[didx.at[0]], ssem).wait()
            @pl.when((r == 0) & (i > 0))
            def _():
                pltpu.sync_copy(
                    dst_h.at[pl.ds(pl.multiple_of(base + i, 8), IB)], didx)
            pltpu.async_copy(buf.at[b], agg_sh.at[didx.at[r]], ssem, add=True)
            @pl.when(i + 1 < nchunks)
            def _():
                rn = lax.rem(i + 1, IB)
                pltpu.async_copy(g_h.at[sidx.at[rn]], buf.at[1 - b], gsem)
        return carry

    lax.fori_loop(0, lax.div(nchunks, 2), body, 0)
    pltpu.make_async_copy(buf.at[1], agg_sh.at[didx.at[0]], ssem).wait()
    plsc.subcore_barrier()
    pltpu.sync_copy(agg_sh.at[pl.ds(s * RPT, RPT)],
                    out_h.at[c, pl.ds(s * RPT, RPT)])


def _sc_aggregate(g, src2, dst2, zeros128):
    return pl.kernel(
        _agg_body,
        out_type=jax.ShapeDtypeStruct((NC, R_PAD, H), jnp.float32),
        mesh=_sc_mesh(),
        scratch_types=[
            pltpu.VMEM_SHARED((R_PAD, H), jnp.float32),
            pltpu.VMEM((2, CHUNK, H), jnp.float32),
            pltpu.VMEM((IB, CHUNK), jnp.int32),
            pltpu.VMEM((IB, CHUNK), jnp.int32),
            pltpu.SemaphoreType.DMA,
            pltpu.SemaphoreType.DMA,
        ],
    )(g, src2, dst2, zeros128)


BLK = 1000
NBLK = N // BLK


def _tc0_body(x_ref, w_ref, b_ref, degs_ref, dinv_ref, g_ref):
    deg = degs_ref[0, :, 0:1] + degs_ref[1, :, 0:1] + 1.0
    dinv = lax.rsqrt(deg)
    dinv_ref[...] = dinv
    h = jnp.dot(x_ref[...], w_ref[...], preferred_element_type=jnp.float32)
    g_ref[...] = (h + b_ref[...]) * dinv


def _tc0(x, w, b, degs):
    return pl.pallas_call(
        _tc0_body,
        grid=(NBLK,),
        in_specs=[
            pl.BlockSpec((BLK, D), lambda i: (i, 0)),
            pl.BlockSpec((D, H), lambda i: (0, 0)),
            pl.BlockSpec((1, H), lambda i: (0, 0)),
            pl.BlockSpec((NC, BLK, H), lambda i: (0, i, 0)),
        ],
        out_specs=[
            pl.BlockSpec((BLK, 1), lambda i: (i, 0)),
            pl.BlockSpec((BLK, H), lambda i: (i, 0)),
        ],
        out_shape=[
            jax.ShapeDtypeStruct((N, 1), jnp.float32),
            jax.ShapeDtypeStruct((N, H), jnp.float32),
        ],
    )(x, w, b, degs)


def _leaky(y):
    return jnp.where(y >= 0, y, ALPHA * y)


def _tc_mid_body(a_ref, gp_ref, dinv_ref, w_ref, b_ref, x_ref, g_ref):
    dinv = dinv_ref[...]
    y = (a_ref[0] + a_ref[1] + gp_ref[...]) * dinv
    xo = _leaky(y)
    x_ref[...] = xo
    h = jnp.dot(xo, w_ref[...], preferred_element_type=jnp.float32)
    g_ref[...] = (h + b_ref[...]) * dinv


def _tc_mid(aggs, g_prev, dinv, w, b):
    return pl.pallas_call(
        _tc_mid_body,
        grid=(NBLK,),
        in_specs=[
            pl.BlockSpec((NC, BLK, H), lambda i: (0, i, 0)),
            pl.BlockSpec((BLK, H), lambda i: (i, 0)),
            pl.BlockSpec((BLK, 1), lambda i: (i, 0)),
            pl.BlockSpec((H, H), lambda i: (0, 0)),
            pl.BlockSpec((1, H), lambda i: (0, 0)),
        ],
        out_specs=[
            pl.BlockSpec((BLK, H), lambda i: (i, 0)),
            pl.BlockSpec((BLK, H), lambda i: (i, 0)),
        ],
        out_shape=[
            jax.ShapeDtypeStruct((N, H), jnp.float32),
            jax.ShapeDtypeStruct((N, H), jnp.float32),
        ],
    )(aggs, g_prev, dinv, w, b)


def _tc_l3_body(a_ref, gp_ref, dinv_ref, x0_ref, w_ref, b_ref,
                x2_ref, x3_ref, g_ref):
    dinv = dinv_ref[...]
    y = (a_ref[0] + a_ref[1] + gp_ref[...]) * dinv
    x2 = _leaky(y)
    x2_ref[...] = x2
    x3 = x0_ref[...] + x2
    x3_ref[...] = x3
    h = jnp.dot(x3, w_ref[...], preferred_element_type=jnp.float32)
    g_ref[...] = (h + b_ref[...]) * dinv


def _tc_l3(aggs, g_prev, dinv, x0, w, b):
    return pl.pallas_call(
        _tc_l3_body,
        grid=(NBLK,),
        in_specs=[
            pl.BlockSpec((NC, BLK, H), lambda i: (0, i, 0)),
            pl.BlockSpec((BLK, H), lambda i: (i, 0)),
            pl.BlockSpec((BLK, 1), lambda i: (i, 0)),
            pl.BlockSpec((BLK, H), lambda i: (i, 0)),
            pl.BlockSpec((H, H), lambda i: (0, 0)),
            pl.BlockSpec((1, H), lambda i: (0, 0)),
        ],
        out_specs=[
            pl.BlockSpec((BLK, H), lambda i: (i, 0)),
            pl.BlockSpec((BLK, H), lambda i: (i, 0)),
            pl.BlockSpec((BLK, H), lambda i: (i, 0)),
        ],
        out_shape=[
            jax.ShapeDtypeStruct((N, H), jnp.float32),
            jax.ShapeDtypeStruct((N, H), jnp.float32),
            jax.ShapeDtypeStruct((N, H), jnp.float32),
        ],
    )(aggs, g_prev, dinv, x0, w, b)


NEG = -3.0e38


def _final_body(a_ref, gp_ref, dinv_ref, x1_ref, x2_ref, x3_ref, batch_ref,
                wm1_ref, bm1_ref, gamma_ref, beta_ref, wm2_ref, bm2_ref,
                out_ref, xt_ref):
    def xt_blk(i, carry):
        sl = pl.ds(i * BLK, BLK)
        y = (a_ref[0, sl, :] + a_ref[1, sl, :] + gp_ref[sl, :]) * dinv_ref[sl, :]
        xt_ref[sl, :] = _leaky(y)
        return carry

    lax.fori_loop(0, NBLK, xt_blk, 0)

    def pool(src_ref):
        def blk(i, acc):
            sl = pl.ds(i * BLK, BLK)
            rows = src_ref[sl, :]
            b = batch_ref[sl, :]
            glo = jnp.min(b)
            ghi = jnp.max(b)

            def gl(g, acc):
                m = b == g
                v = jnp.max(jnp.where(m, rows, NEG), axis=0, keepdims=True)
                sel = lax.broadcasted_iota(jnp.int32, (G, 1), 0) == g
                return jnp.where(sel, jnp.maximum(acc, v), acc)

            return lax.fori_loop(glo, ghi + 1, gl, acc)

        return lax.fori_loop(0, NBLK, blk, jnp.full((G, H), NEG, jnp.float32))

    pooled = jnp.concatenate(
        [pool(xt_ref), pool(x1_ref), pool(x2_ref), pool(x3_ref)], axis=1)

    h = jnp.dot(pooled, wm1_ref[...], preferred_element_type=jnp.float32)
    h = h + bm1_ref[...]
    mean = jnp.mean(h, axis=0, keepdims=True)
    var = jnp.mean((h - mean) ** 2, axis=0, keepdims=True)
    h = (h - mean) * lax.rsqrt(var + 1e-5) * gamma_ref[...] + beta_ref[...]
    h = jnp.maximum(h, 0.0)
    out_ref[...] = jnp.dot(h, wm2_ref[...], preferred_element_type=jnp.float32) + bm2_ref[...]


def _tc_final(aggs, g3, dinv, x1, x2, x3, batch2, wm1, bm1, gamma, beta, wm2, bm2):
    return pl.pallas_call(
        _final_body,
        out_shape=jax.ShapeDtypeStruct((G, C), jnp.float32),
        scratch_shapes=[pltpu.VMEM((N, H), jnp.float32)],
    )(aggs, g3, dinv, x1, x2, x3, batch2,
      wm1, bm1, gamma, beta, wm2, bm2)


def kernel(x, edge_index, batch, W0, b0, W1, b1, W2, b2, W3, b3,
           Wm1, bm1, gamma, beta, Wm2, bm2):
    pad = E_PAD - E
    src2 = jnp.concatenate(
        [edge_index[0], jnp.zeros((pad,), jnp.int32)]).reshape(E_PAD // CHUNK, CHUNK)
    dst2 = jnp.concatenate(
        [edge_index[1], jnp.full((pad,), N, jnp.int32)]).reshape(E_PAD // CHUNK, CHUNK)
    zeros128 = jnp.zeros((R_PAD, H), jnp.float32)
    ones128 = jnp.ones((CHUNK, H), jnp.float32)
    batch2 = batch.reshape(N, 1)
    b0r, b1r, b2r, b3r = (v.reshape(1, H) for v in (b0, b1, b2, b3))
    bm1r = bm1.reshape(1, H)
    gammar = gamma.reshape(1, H)
    betar = beta.reshape(1, H)
    bm2r = bm2.reshape(1, C)

    degs = _sc_degree(dst2, ones128, zeros128)
    dinv, g0 = _tc0(x, W0, b0r, degs)
    a0 = _sc_aggregate(g0, src2, dst2, zeros128)
    x0, g1 = _tc_mid(a0, g0, dinv, W1, b1r)
    a1 = _sc_aggregate(g1, src2, dst2, zeros128)
    x1, g2 = _tc_mid(a1, g1, dinv, W2, b2r)
    a2 = _sc_aggregate(g2, src2, dst2, zeros128)
    x2, x3, g3 = _tc_l3(a2, g2, dinv, x0, W3, b3r)
    a3 = _sc_aggregate(g3, src2, dst2, zeros128)
    return _tc_final(a3, g3, dinv, x1, x2, x3, batch2,
                     Wm1, bm1r, gammar, betar, Wm2, bm2r)
